# baseline (speedup 1.0000x reference)
import jax
import jax.numpy as jnp
from jax.experimental import pallas as pl
from jax.experimental.pallas import tpu as pltpu

_B, _P, _D = 8192, 2048, 512
_BM = 256
_PT = 512


def _body(x_ref, pt_ref, wt_ref, b_ref, o_ref):
    x = x_ref[...]
    in_sq = jnp.sum(x * x, axis=1, keepdims=True)
    acc = jnp.zeros((_BM, _D), dtype=jnp.float32)
    for j in range(_P // _PT):
        ptj = pt_ref[:, j * _PT:(j + 1) * _PT]
        pr_sq = jnp.sum(ptj * ptj, axis=0, keepdims=True)
        cross = jnp.dot(x, ptj, preferred_element_type=jnp.float32)
        d2 = jnp.maximum(in_sq + pr_sq - 2.0 * cross, 0.0)
        dist = jnp.sqrt(d2)
        acc = acc + jnp.dot(dist, wt_ref[j * _PT:(j + 1) * _PT, :],
                            preferred_element_type=jnp.float32)
    o_ref[...] = acc + b_ref[...]


def kernel(input, prototypes, W, b):
    pt = prototypes.T
    wt = W.T
    b2 = b.reshape(1, _D)
    return pl.pallas_call(
        _body,
        out_shape=jax.ShapeDtypeStruct((_B, _D), jnp.float32),
        grid=(_B // _BM,),
        in_specs=[
            pl.BlockSpec((_BM, _D), lambda i: (i, 0)),
            pl.BlockSpec((_D, _P), lambda i: (0, 0)),
            pl.BlockSpec((_P, _D), lambda i: (0, 0)),
            pl.BlockSpec((1, _D), lambda i: (0, 0)),
        ],
        out_specs=pl.BlockSpec((_BM, _D), lambda i: (i, 0)),
        compiler_params=pltpu.CompilerParams(
            dimension_semantics=("parallel",),
            vmem_limit_bytes=48 * 1024 * 1024,
        ),
        name="proto_classifier_fused",
    )(input, pt, wt, b2)


# speedup vs baseline: 1.0158x; 1.0158x over previous
import jax
import jax.numpy as jnp
from jax.experimental import pallas as pl
from jax.experimental.pallas import tpu as pltpu

_B, _P, _D = 8192, 2048, 512
_BM = 256
_PT = 512


def _body(x_ref, pt_ref, wt_ref, b_ref, o_ref, psq_ref):
    @pl.when(pl.program_id(0) == 0)
    def _():
        p = pt_ref[...]
        psq_ref[...] = jnp.sum(p * p, axis=0, keepdims=True)

    x = x_ref[...]
    in_sq = jnp.sum(x * x, axis=1, keepdims=True)
    xs = -2.0 * x
    acc = jnp.zeros((_BM, _D), dtype=jnp.float32)
    for j in range(_P // _PT):
        sl = slice(j * _PT, (j + 1) * _PT)
        cross = jnp.dot(xs, pt_ref[:, sl],
                        preferred_element_type=jnp.float32)
        d2 = (in_sq + psq_ref[:, sl]) + cross
        d2c = jnp.maximum(d2, 1e-30)
        dist = d2c * jax.lax.rsqrt(d2c)
        acc = acc + jnp.dot(dist, wt_ref[sl, :],
                            preferred_element_type=jnp.float32)
    o_ref[...] = acc + b_ref[...]


def kernel(input, prototypes, W, b):
    pt = prototypes.T
    wt = W.T
    b2 = b.reshape(1, _D)
    return pl.pallas_call(
        _body,
        out_shape=jax.ShapeDtypeStruct((_B, _D), jnp.float32),
        grid=(_B // _BM,),
        in_specs=[
            pl.BlockSpec((_BM, _D), lambda i: (i, 0)),
            pl.BlockSpec((_D, _P), lambda i: (0, 0)),
            pl.BlockSpec((_P, _D), lambda i: (0, 0)),
            pl.BlockSpec((1, _D), lambda i: (0, 0)),
        ],
        out_specs=pl.BlockSpec((_BM, _D), lambda i: (i, 0)),
        scratch_shapes=[pltpu.VMEM((1, _P), jnp.float32)],
        compiler_params=pltpu.CompilerParams(
            dimension_semantics=("arbitrary",),
            vmem_limit_bytes=48 * 1024 * 1024,
        ),
        name="proto_classifier_fused",
    )(input, pt, wt, b2)


# speedup vs baseline: 1.0384x; 1.0223x over previous
import jax
import jax.numpy as jnp
from jax.experimental import pallas as pl
from jax.experimental.pallas import tpu as pltpu

_B, _P, _D = 8192, 2048, 512
_BM = 256
_PT = 512


def _body(x_ref, pt_ref, wt_ref, b_ref, o_ref, psq_ref):
    @pl.when(pl.program_id(0) == 0)
    def _():
        p = pt_ref[...].astype(jnp.float32)
        psq_ref[...] = jnp.sum(p * p, axis=0, keepdims=True)

    x = x_ref[...]
    in_sq = jnp.sum(x * x, axis=1, keepdims=True)
    xs = (-2.0 * x).astype(jnp.bfloat16)
    acc = jnp.zeros((_BM, _D), dtype=jnp.float32)
    for j in range(_P // _PT):
        sl = slice(j * _PT, (j + 1) * _PT)
        cross = jnp.dot(xs, pt_ref[:, sl],
                        preferred_element_type=jnp.float32)
        d2 = (in_sq + psq_ref[:, sl]) + cross
        d2c = jnp.maximum(d2, 1e-30)
        dist = (d2c * jax.lax.rsqrt(d2c)).astype(jnp.bfloat16)
        acc = acc + jnp.dot(dist, wt_ref[sl, :],
                            preferred_element_type=jnp.float32)
    o_ref[...] = acc + b_ref[...]


def kernel(input, prototypes, W, b):
    pt = prototypes.T.astype(jnp.bfloat16)
    wt = W.T.astype(jnp.bfloat16)
    b2 = b.reshape(1, _D)
    return pl.pallas_call(
        _body,
        out_shape=jax.ShapeDtypeStruct((_B, _D), jnp.float32),
        grid=(_B // _BM,),
        in_specs=[
            pl.BlockSpec((_BM, _D), lambda i: (i, 0)),
            pl.BlockSpec((_D, _P), lambda i: (0, 0)),
            pl.BlockSpec((_P, _D), lambda i: (0, 0)),
            pl.BlockSpec((1, _D), lambda i: (0, 0)),
        ],
        out_specs=pl.BlockSpec((_BM, _D), lambda i: (i, 0)),
        scratch_shapes=[pltpu.VMEM((1, _P), jnp.float32)],
        compiler_params=pltpu.CompilerParams(
            dimension_semantics=("arbitrary",),
            vmem_limit_bytes=48 * 1024 * 1024,
        ),
        name="proto_classifier_fused",
    )(input, pt, wt, b2)


# speedup vs baseline: 1.1213x; 1.0798x over previous
import jax
import jax.numpy as jnp
from jax.experimental import pallas as pl
from jax.experimental.pallas import tpu as pltpu

_B, _P, _D = 8192, 2048, 512
_BM = 1024
_SM = 256
_PT = 512


def _body(x_ref, pt_ref, wt_ref, b_ref, o_ref, psq_ref):
    @pl.when(pl.program_id(0) == 0)
    def _():
        p = pt_ref[...].astype(jnp.float32)
        psq_ref[...] = jnp.sum(p * p, axis=0, keepdims=True)

    for mi in range(_BM // _SM):
        x = x_ref[mi * _SM:(mi + 1) * _SM, :]
        in_sq = jnp.sum(x * x, axis=1, keepdims=True)
        xs = (-2.0 * x).astype(jnp.bfloat16)
        acc = jnp.zeros((_SM, _D), dtype=jnp.float32)
        for j in range(_P // _PT):
            sl = slice(j * _PT, (j + 1) * _PT)
            cross = jnp.dot(xs, pt_ref[:, sl],
                            preferred_element_type=jnp.float32)
            d2 = (in_sq + psq_ref[:, sl]) + cross
            d2c = jnp.maximum(d2, 1e-30)
            dist = (d2c * jax.lax.rsqrt(d2c)).astype(jnp.bfloat16)
            acc = acc + jnp.dot(dist, wt_ref[sl, :],
                                preferred_element_type=jnp.float32)
        o_ref[mi * _SM:(mi + 1) * _SM, :] = acc + b_ref[...]


def kernel(input, prototypes, W, b):
    pt = prototypes.T.astype(jnp.bfloat16)
    wt = W.T.astype(jnp.bfloat16)
    b2 = b.reshape(1, _D)
    return pl.pallas_call(
        _body,
        out_shape=jax.ShapeDtypeStruct((_B, _D), jnp.float32),
        grid=(_B // _BM,),
        in_specs=[
            pl.BlockSpec((_BM, _D), lambda i: (i, 0)),
            pl.BlockSpec((_D, _P), lambda i: (0, 0)),
            pl.BlockSpec((_P, _D), lambda i: (0, 0)),
            pl.BlockSpec((1, _D), lambda i: (0, 0)),
        ],
        out_specs=pl.BlockSpec((_BM, _D), lambda i: (i, 0)),
        scratch_shapes=[pltpu.VMEM((1, _P), jnp.float32)],
        compiler_params=pltpu.CompilerParams(
            dimension_semantics=("arbitrary",),
            vmem_limit_bytes=48 * 1024 * 1024,
        ),
        name="proto_classifier_fused",
    )(input, pt, wt, b2)


# speedup vs baseline: 1.3331x; 1.1890x over previous
import jax
import jax.numpy as jnp
from jax.experimental import pallas as pl
from jax.experimental.pallas import tpu as pltpu

_B, _P, _D = 8192, 2048, 512
_BM = 1024
_SM = 256
_PT = 256


def _body(x_ref, pt_ref, wt_ref, b_ref, o_ref, psq_ref, ds_ref):
    @pl.when(pl.program_id(0) == 0)
    def _():
        p = pt_ref[...].astype(jnp.float32)
        psq_ref[...] = jnp.sum(p * p, axis=0, keepdims=True)

    for mi in range(_BM // _SM):
        x = x_ref[mi * _SM:(mi + 1) * _SM, :]
        in_sq = jnp.sum(x * x, axis=1, keepdims=True)
        xs = (-2.0 * x).astype(jnp.bfloat16)
        for j in range(_P // _PT):
            sl = slice(j * _PT, (j + 1) * _PT)
            cross = jnp.dot(xs, pt_ref[:, sl],
                            preferred_element_type=jnp.float32)
            d2 = (in_sq + psq_ref[:, sl]) + cross
            d2c = jnp.maximum(d2, 1e-30)
            ds_ref[mi, :, sl] = (d2c * jax.lax.rsqrt(d2c)).astype(jnp.bfloat16)
        acc = jnp.dot(ds_ref[mi], wt_ref[...],
                      preferred_element_type=jnp.float32)
        o_ref[mi * _SM:(mi + 1) * _SM, :] = acc + b_ref[...]


def kernel(input, prototypes, W, b):
    pt = prototypes.T.astype(jnp.bfloat16)
    wt = W.T.astype(jnp.bfloat16)
    b2 = b.reshape(1, _D)
    return pl.pallas_call(
        _body,
        out_shape=jax.ShapeDtypeStruct((_B, _D), jnp.float32),
        grid=(_B // _BM,),
        in_specs=[
            pl.BlockSpec((_BM, _D), lambda i: (i, 0)),
            pl.BlockSpec((_D, _P), lambda i: (0, 0)),
            pl.BlockSpec((_P, _D), lambda i: (0, 0)),
            pl.BlockSpec((1, _D), lambda i: (0, 0)),
        ],
        out_specs=pl.BlockSpec((_BM, _D), lambda i: (i, 0)),
        scratch_shapes=[
            pltpu.VMEM((1, _P), jnp.float32),
            pltpu.VMEM((_BM // _SM, _SM, _P), jnp.bfloat16),
        ],
        compiler_params=pltpu.CompilerParams(
            dimension_semantics=("arbitrary",),
            vmem_limit_bytes=48 * 1024 * 1024,
        ),
        name="proto_classifier_fused",
    )(input, pt, wt, b2)
